# Initial kernel scaffold; baseline (speedup 1.0000x reference)
import jax
import jax.numpy as jnp
from jax.experimental import pallas as pl


def kernel(x, batch, edge_index, Wp1, bp1, Wq1, bq1, Wr1, br1, gat_W, att_src, att_dst, gat_b, gn_w, gn_b, gn_ms, Wp2, bp2, Wq2, bq2, Wr2, br2):
    raise NotImplementedError("write your pallas kernel here")


# speedup vs baseline: 1.0053x; 1.0053x over previous
import functools

import jax
import jax.numpy as jnp
from jax.experimental import pallas as pl
from jax.experimental.pallas import tpu as pltpu

NEG = 0.2
EPS = 1e-5


def _tconv_body(Tout, Kk, x_ref, wp_ref, bp_ref, wq_ref, bq_ref, wr_ref, br_ref, o_ref):
    xb = x_ref[0]
    for to in range(Tout):
        p = bp_ref[...].astype(jnp.float32)
        q = bq_ref[...].astype(jnp.float32)
        r = br_ref[...].astype(jnp.float32)
        for k in range(Kk):
            xt = xb[to + k]
            p = p + jnp.dot(xt, wp_ref[k], preferred_element_type=jnp.float32)
            q = q + jnp.dot(xt, wq_ref[k], preferred_element_type=jnp.float32)
            r = r + jnp.dot(xt, wr_ref[k], preferred_element_type=jnp.float32)
        o_ref[0, to] = jnp.maximum(p * jax.nn.sigmoid(q) + r, 0.0)


def _tconv(x, Wp, bp, Wq, bq, Wr, br):
    B_, Tin, N_, C = x.shape
    Kk, _, H_ = Wp.shape
    Tout = Tin - Kk + 1
    BN = 1000
    grid = (B_, N_ // BN)
    return pl.pallas_call(
        functools.partial(_tconv_body, Tout, Kk),
        grid=grid,
        in_specs=[
            pl.BlockSpec((1, Tin, BN, C), lambda b, n: (b, 0, n, 0)),
            pl.BlockSpec((Kk, C, H_), lambda b, n: (0, 0, 0)),
            pl.BlockSpec((H_,), lambda b, n: (0,)),
            pl.BlockSpec((Kk, C, H_), lambda b, n: (0, 0, 0)),
            pl.BlockSpec((H_,), lambda b, n: (0,)),
            pl.BlockSpec((Kk, C, H_), lambda b, n: (0, 0, 0)),
            pl.BlockSpec((H_,), lambda b, n: (0,)),
        ],
        out_specs=pl.BlockSpec((1, Tout, BN, H_), lambda b, n: (b, 0, n, 0)),
        out_shape=jax.ShapeDtypeStruct((B_, Tout, N_, H_), jnp.float32),
    )(x, Wp, bp, Wq, bq, Wr, br)


def _gat(x, edge_index, W, a_src, a_dst, b):
    h = x @ W
    src = edge_index[0]
    dst = edge_index[1]
    n = x.shape[0]
    al = jax.nn.leaky_relu((h @ a_src)[src] + (h @ a_dst)[dst], NEG)
    amax = jax.ops.segment_max(al, dst, num_segments=n)
    ex = jnp.exp(al - amax[dst])
    den = jax.ops.segment_sum(ex, dst, num_segments=n)
    alpha = ex / (den[dst] + 1e-16)
    out = jax.ops.segment_sum(h[src] * alpha[:, None], dst, num_segments=n)
    return out + b


def _graph_norm(x, batch, w, b, ms, num_graphs):
    ones = jnp.ones((x.shape[0],), x.dtype)
    cnt = jnp.maximum(jax.ops.segment_sum(ones, batch, num_segments=num_graphs), 1.0)
    mean = jax.ops.segment_sum(x, batch, num_segments=num_graphs) / cnt[:, None]
    out = x - mean[batch] * ms
    var = jax.ops.segment_sum(out * out, batch, num_segments=num_graphs) / cnt[:, None]
    std = jnp.sqrt(var + EPS)
    return w * out / std[batch] + b


def kernel(x, batch, edge_index, Wp1, bp1, Wq1, bq1, Wr1, br1, gat_W, att_src,
           att_dst, gat_b, gn_w, gn_b, gn_ms, Wp2, bp2, Wq2, bq2, Wr2, br2):
    out0 = _tconv(x, Wp1, bp1, Wq1, bq1, Wr1, br1)
    b_, t_, n_, h_ = out0.shape

    def step(xi):
        xi = xi.reshape(b_ * n_, h_)
        xi = _gat(xi, edge_index, gat_W, att_src, att_dst, gat_b)
        xi = _graph_norm(xi, batch, gn_w, gn_b, gn_ms, b_)
        return xi.reshape(b_, n_, h_)

    out = jax.vmap(step, in_axes=1, out_axes=1)(out0)
    out = jax.nn.relu(out)
    return _tconv(out, Wp2, bp2, Wq2, bq2, Wr2, br2)


# speedup vs baseline: 9.2401x; 9.1910x over previous
import functools

import jax
import jax.numpy as jnp
from jax import lax
from jax.experimental import pallas as pl
from jax.experimental.pallas import tpu as pltpu
from jax.experimental.pallas import tpu_sc as plsc

NEG = 0.2
EPS = 1e-5

NN = 20000
NE = 320000
NT = 6
HH = 64
WACC = 72
EK = 80
WCH = 80
NWCH = NN // WCH
ZCH = 160
NZCH = NN // ZCH


def _tconv_body(Tout, Kk, x_ref, wp_ref, bp_ref, wq_ref, bq_ref, wr_ref, br_ref, o_ref):
    xb = x_ref[0]
    for to in range(Tout):
        p = bp_ref[...].astype(jnp.float32)
        q = bq_ref[...].astype(jnp.float32)
        r = br_ref[...].astype(jnp.float32)
        for k in range(Kk):
            xt = xb[to + k]
            p = p + jnp.dot(xt, wp_ref[k], preferred_element_type=jnp.float32)
            q = q + jnp.dot(xt, wq_ref[k], preferred_element_type=jnp.float32)
            r = r + jnp.dot(xt, wr_ref[k], preferred_element_type=jnp.float32)
        o_ref[0, to] = jnp.maximum(p * jax.nn.sigmoid(q) + r, 0.0)


def _tconv(x, Wp, bp, Wq, bq, Wr, br):
    B_, Tin, N_, C = x.shape
    Kk, _, H_ = Wp.shape
    Tout = Tin - Kk + 1
    BN = 1000
    grid = (B_, N_ // BN)
    return pl.pallas_call(
        functools.partial(_tconv_body, Tout, Kk),
        grid=grid,
        in_specs=[
            pl.BlockSpec((1, Tin, BN, C), lambda b, n: (b, 0, n, 0)),
            pl.BlockSpec((Kk, C, H_), lambda b, n: (0, 0, 0)),
            pl.BlockSpec((H_,), lambda b, n: (0,)),
            pl.BlockSpec((Kk, C, H_), lambda b, n: (0, 0, 0)),
            pl.BlockSpec((H_,), lambda b, n: (0,)),
            pl.BlockSpec((Kk, C, H_), lambda b, n: (0, 0, 0)),
            pl.BlockSpec((H_,), lambda b, n: (0,)),
        ],
        out_specs=pl.BlockSpec((1, Tout, BN, H_), lambda b, n: (b, 0, n, 0)),
        out_shape=jax.ShapeDtypeStruct((B_, Tout, N_, H_), jnp.float32),
    )(x, Wp, bp, Wq, bq, Wr, br)


def _pre_body(x_ref, w_ref, asd_ref, h_ref, aux_ref):
    xb = x_ref[0]
    h = jnp.dot(xb, w_ref[...], preferred_element_type=jnp.float32)
    aux = jnp.dot(h, asd_ref[...], preferred_element_type=jnp.float32)
    s = aux[:, 0:1]
    z = jnp.zeros((xb.shape[0], WACC - HH - 1), jnp.float32)
    h_ref[0, 0] = jnp.concatenate([h[:, :HH], s, z], axis=1)
    h_ref[0, 1] = jnp.concatenate([h[:, HH:], s, z], axis=1)
    aux_ref[0] = aux


def _pre(xg, W, asd):
    T_, NN_, C = xg.shape
    BN = 2000
    grid = (T_, NN_ // BN)
    return pl.pallas_call(
        _pre_body,
        grid=grid,
        in_specs=[
            pl.BlockSpec((1, BN, C), lambda t, n: (t, n, 0)),
            pl.BlockSpec((C, C), lambda t, n: (0, 0)),
            pl.BlockSpec((C, 16), lambda t, n: (0, 0)),
        ],
        out_specs=[
            pl.BlockSpec((1, 2, BN, WACC), lambda t, n: (t, 0, n, 0)),
            pl.BlockSpec((1, BN, 16), lambda t, n: (t, n, 0)),
        ],
        out_shape=[
            jax.ShapeDtypeStruct((T_, 2, NN_, WACC), jnp.float32),
            jax.ShapeDtypeStruct((T_, NN_, 16), jnp.float32),
        ],
    )(xg, W, asd)


def _gnorm_body(y_ref, bc_ref, w_ref, b_ref, ms_ref, o_ref, acc_ref):
    ph = pl.program_id(1)
    nb = pl.program_id(2)
    yb = y_ref[0]
    m1 = bc_ref[...]
    m0 = 1.0 - m1

    @pl.when((ph == 0) & (nb == 0))
    def _():
        acc_ref[...] = jnp.zeros_like(acc_ref)

    @pl.when(ph == 0)
    def _():
        acc_ref[0:1] += jnp.sum(yb * m0, axis=0, keepdims=True)
        acc_ref[1:2] += jnp.sum(yb * m1, axis=0, keepdims=True)
        acc_ref[2:3] += jnp.full((1, yb.shape[1]), 1.0) * jnp.sum(m0)
        acc_ref[3:4] += jnp.full((1, yb.shape[1]), 1.0) * jnp.sum(m1)

    c0 = jnp.maximum(acc_ref[2:3], 1.0)
    c1 = jnp.maximum(acc_ref[3:4], 1.0)
    mean0 = acc_ref[0:1] / c0
    mean1 = acc_ref[1:2] / c1
    ctr = yb - (m0 * mean0 + m1 * mean1) * ms_ref[...]

    @pl.when(ph == 1)
    def _():
        sq = ctr * ctr
        acc_ref[4:5] += jnp.sum(sq * m0, axis=0, keepdims=True)
        acc_ref[5:6] += jnp.sum(sq * m1, axis=0, keepdims=True)

    s0 = jnp.sqrt(acc_ref[4:5] / c0 + EPS)
    s1 = jnp.sqrt(acc_ref[5:6] / c1 + EPS)
    o = w_ref[...] * ctr / (m0 * s0 + m1 * s1) + b_ref[...]
    o_ref[0] = jnp.maximum(o, 0.0)


def _gnorm(y, bcol, w, b, ms):
    T_, NN_, H_ = y.shape
    BN = 2000
    return pl.pallas_call(
        _gnorm_body,
        grid=(T_, 3, NN_ // BN),
        in_specs=[
            pl.BlockSpec((1, BN, H_), lambda t, p, n: (t, n, 0)),
            pl.BlockSpec((BN, 1), lambda t, p, n: (n, 0)),
            pl.BlockSpec((1, H_), lambda t, p, n: (0, 0)),
            pl.BlockSpec((1, H_), lambda t, p, n: (0, 0)),
            pl.BlockSpec((1, H_), lambda t, p, n: (0, 0)),
        ],
        out_specs=pl.BlockSpec((1, BN, H_), lambda t, p, n: (t, n, 0)),
        out_shape=jax.ShapeDtypeStruct((T_, NN_, H_), jnp.float32),
        scratch_shapes=[pltpu.VMEM((8, H_), jnp.float32)],
    )(y, bcol, w, b, ms)


def _gat_sc_body(h_hbm, d_hbm, src_hbm, dst_hbm, zeros_hbm, bias_hbm,
                 out_hbm, d_v, src_blk, dst_blk, gidx, exbuf, sc_v, obuf,
                 b_v, num_sh, sem):
    c = lax.axis_index("c")
    sid = lax.axis_index("s")
    pltpu.sync_copy(bias_hbm.at[pl.ds(c * HH, HH)], b_v)
    nzch = (NZCH - sid + 15) // 16
    nwch = (NWCH - sid + 15) // 16
    lanes = lax.iota(jnp.int32, 16)

    def t_body(t, carry):
        tc = t * 2 + c
        row_off = tc * NN

        def z_body(i, _):
            nb = (sid + i * 16) * ZCH
            pltpu.sync_copy(zeros_hbm, num_sh.at[pl.ds(nb, ZCH)])
            return 0
        lax.fori_loop(0, nzch, z_body, 0)
        pltpu.sync_copy(d_hbm.at[pl.ds(t * NN, NN)], d_v)
        plsc.subcore_barrier()

        def e_body(i, _):
            base = sid * (NE // 16) + i * EK
            pltpu.sync_copy(src_hbm.at[pl.ds(base, EK)], src_blk)
            pltpu.sync_copy(dst_hbm.at[pl.ds(base, EK)], dst_blk)
            for q in range(EK // 16):
                sl = pl.ds(q * 16, 16)
                gidx[sl] = src_blk[sl] + row_off
            pltpu.async_copy(h_hbm.at[gidx], sc_v, sem).wait()
            col64 = jnp.full((16,), HH, jnp.int32)
            for q in range(EK // 16):
                sl = pl.ds(q * 16, 16)
                rows = lanes + (q * 16)
                sv = plsc.load_gather(sc_v, [rows, col64])
                dv = plsc.load_gather(d_v, [dst_blk[sl]])
                al = sv + dv
                al = jnp.where(al >= 0.0, al, al * NEG)
                exbuf[sl] = jnp.exp(al)
            for q in range(EK // 16):
                exq = exbuf[pl.ds(q * 16, 16)]
                for l in range(16):
                    e = q * 16 + l
                    exs = jnp.full((16,), exq[l], jnp.float32)
                    for j in range(4):
                        sj = pl.ds(j * 16, 16)
                        sc_v[e, sj] = sc_v[e, sj] * exs
                    v56 = sc_v[e, pl.ds(56, 16)]
                    sc_v[e, pl.ds(56, 16)] = jnp.where(
                        lanes < 8, v56,
                        jnp.where(lanes == 8, exs, jnp.zeros((16,), jnp.float32)))
            pltpu.sync_copy(sc_v, num_sh.at[dst_blk], add=True)
            return 0
        lax.fori_loop(0, NE // 16 // EK, e_body, 0)
        plsc.subcore_barrier()

        def w_body(i, _):
            nb = (sid + i * 16) * WCH
            pltpu.sync_copy(num_sh.at[pl.ds(nb, WCH)], sc_v)
            for e in range(WCH):
                denv = sc_v[e, pl.ds(56, 16)]
                dens = jnp.full((16,), denv[8] + 1e-16, jnp.float32)
                for j in range(4):
                    sj = pl.ds(j * 16, 16)
                    obuf[e, sj] = sc_v[e, sj] / dens + b_v[sj]
            pltpu.sync_copy(obuf, out_hbm.at[pl.ds(row_off + nb, WCH)])
            return 0
        lax.fori_loop(0, nwch, w_body, 0)
        plsc.subcore_barrier()
        return 0

    lax.fori_loop(0, NT, t_body, 0)


_gat_sc = functools.partial(
    pl.kernel,
    mesh=plsc.VectorSubcoreMesh(core_axis_name="c", subcore_axis_name="s"),
    out_type=jax.ShapeDtypeStruct((2 * NT * NN, HH), jnp.float32),
    compiler_params=pltpu.CompilerParams(
        needs_layout_passes=False, use_tc_tiling_on_sc=False),
    scratch_types=[
        pltpu.VMEM((NN,), jnp.float32),
        pltpu.VMEM((EK,), jnp.int32),
        pltpu.VMEM((EK,), jnp.int32),
        pltpu.VMEM((EK,), jnp.int32),
        pltpu.VMEM((EK,), jnp.float32),
        pltpu.VMEM((EK, WACC), jnp.float32),
        pltpu.VMEM((WCH, HH), jnp.float32),
        pltpu.VMEM((HH,), jnp.float32),
        pltpu.VMEM_SHARED((NN, WACC), jnp.float32),
        pltpu.SemaphoreType.DMA,
    ],
)(_gat_sc_body)


def kernel(x, batch, edge_index, Wp1, bp1, Wq1, bq1, Wr1, br1, gat_W, att_src,
           att_dst, gat_b, gn_w, gn_b, gn_ms, Wp2, bp2, Wq2, bq2, Wr2, br2):
    out0 = _tconv(x, Wp1, bp1, Wq1, bq1, Wr1, br1)
    b_, t_, n_, h_ = out0.shape

    xg = out0.transpose(1, 0, 2, 3).reshape(t_, b_ * n_, h_)
    asd = jnp.zeros((h_, 16), jnp.float32).at[:, 0].set(att_src).at[:, 1].set(att_dst)
    h_all, aux = _pre(xg, gat_W, asd)
    d_all = aux[:, :, 1].reshape(-1)
    h_flat = h_all.reshape(2 * t_ * b_ * n_, WACC)
    src = edge_index[0].astype(jnp.int32)
    dst = edge_index[1].astype(jnp.int32)
    zeros = jnp.zeros((ZCH, WACC), jnp.float32)

    y = _gat_sc(h_flat, d_all, src, dst, zeros, gat_b)
    y = y.reshape(t_, 2, b_ * n_, HH).transpose(0, 2, 1, 3).reshape(t_, b_ * n_, h_)

    bcol = batch.astype(jnp.float32).reshape(b_ * n_, 1)
    y = _gnorm(y, bcol, gn_w.reshape(1, h_), gn_b.reshape(1, h_), gn_ms.reshape(1, h_))
    y = y.reshape(t_, b_, n_, h_).transpose(1, 0, 2, 3)
    return _tconv(y, Wp2, bp2, Wq2, bq2, Wr2, br2)


# speedup vs baseline: 12.5409x; 1.3572x over previous
import functools

import jax
import jax.numpy as jnp
from jax import lax
from jax.experimental import pallas as pl
from jax.experimental.pallas import tpu as pltpu
from jax.experimental.pallas import tpu_sc as plsc

NEG = 0.2
EPS = 1e-5

NN = 20000
NE = 320000
NT = 6
HH = 64
WACC = 72
EK = 80
WCH = 40
NWCH = NN // WCH
ZCH = 160
NZCH = NN // ZCH


def _tconv_body(Tout, Kk, x_ref, wp_ref, bp_ref, wq_ref, bq_ref, wr_ref, br_ref, o_ref):
    xb = x_ref[0]
    for to in range(Tout):
        p = bp_ref[...].astype(jnp.float32)
        q = bq_ref[...].astype(jnp.float32)
        r = br_ref[...].astype(jnp.float32)
        for k in range(Kk):
            xt = xb[to + k]
            p = p + jnp.dot(xt, wp_ref[k], preferred_element_type=jnp.float32)
            q = q + jnp.dot(xt, wq_ref[k], preferred_element_type=jnp.float32)
            r = r + jnp.dot(xt, wr_ref[k], preferred_element_type=jnp.float32)
        o_ref[0, to] = jnp.maximum(p * jax.nn.sigmoid(q) + r, 0.0)


def _tconv(x, Wp, bp, Wq, bq, Wr, br):
    B_, Tin, N_, C = x.shape
    Kk, _, H_ = Wp.shape
    Tout = Tin - Kk + 1
    BN = 1000
    grid = (B_, N_ // BN)
    return pl.pallas_call(
        functools.partial(_tconv_body, Tout, Kk),
        grid=grid,
        in_specs=[
            pl.BlockSpec((1, Tin, BN, C), lambda b, n: (b, 0, n, 0)),
            pl.BlockSpec((Kk, C, H_), lambda b, n: (0, 0, 0)),
            pl.BlockSpec((H_,), lambda b, n: (0,)),
            pl.BlockSpec((Kk, C, H_), lambda b, n: (0, 0, 0)),
            pl.BlockSpec((H_,), lambda b, n: (0,)),
            pl.BlockSpec((Kk, C, H_), lambda b, n: (0, 0, 0)),
            pl.BlockSpec((H_,), lambda b, n: (0,)),
        ],
        out_specs=pl.BlockSpec((1, Tout, BN, H_), lambda b, n: (b, 0, n, 0)),
        out_shape=jax.ShapeDtypeStruct((B_, Tout, N_, H_), jnp.float32),
    )(x, Wp, bp, Wq, bq, Wr, br)


def _pre_body(x_ref, w_ref, asd_ref, h_ref, aux_ref):
    xb = x_ref[0]
    h = jnp.dot(xb, w_ref[...], preferred_element_type=jnp.float32)
    aux = jnp.dot(h, asd_ref[...], preferred_element_type=jnp.float32)
    s = aux[:, 0:1]
    z = jnp.zeros((xb.shape[0], WACC - HH - 1), jnp.float32)
    h_ref[0, 0] = jnp.concatenate([h[:, :HH], s, z], axis=1)
    h_ref[0, 1] = jnp.concatenate([h[:, HH:], s, z], axis=1)
    aux_ref[0] = aux


def _pre(xg, W, asd):
    T_, NN_, C = xg.shape
    BN = 2000
    grid = (T_, NN_ // BN)
    return pl.pallas_call(
        _pre_body,
        grid=grid,
        in_specs=[
            pl.BlockSpec((1, BN, C), lambda t, n: (t, n, 0)),
            pl.BlockSpec((C, C), lambda t, n: (0, 0)),
            pl.BlockSpec((C, 16), lambda t, n: (0, 0)),
        ],
        out_specs=[
            pl.BlockSpec((1, 2, BN, WACC), lambda t, n: (t, 0, n, 0)),
            pl.BlockSpec((1, BN, 16), lambda t, n: (t, n, 0)),
        ],
        out_shape=[
            jax.ShapeDtypeStruct((T_, 2, NN_, WACC), jnp.float32),
            jax.ShapeDtypeStruct((T_, NN_, 16), jnp.float32),
        ],
    )(xg, W, asd)


def _gnorm_body(y_ref, bc_ref, w_ref, b_ref, ms_ref, o_ref, acc_ref):
    ph = pl.program_id(1)
    nb = pl.program_id(2)
    yb = y_ref[0]
    m1 = bc_ref[...]
    m0 = 1.0 - m1

    @pl.when((ph == 0) & (nb == 0))
    def _():
        acc_ref[...] = jnp.zeros_like(acc_ref)

    @pl.when(ph == 0)
    def _():
        acc_ref[0:1] += jnp.sum(yb * m0, axis=0, keepdims=True)
        acc_ref[1:2] += jnp.sum(yb * m1, axis=0, keepdims=True)
        acc_ref[2:3] += jnp.full((1, yb.shape[1]), 1.0) * jnp.sum(m0)
        acc_ref[3:4] += jnp.full((1, yb.shape[1]), 1.0) * jnp.sum(m1)

    c0 = jnp.maximum(acc_ref[2:3], 1.0)
    c1 = jnp.maximum(acc_ref[3:4], 1.0)
    mean0 = acc_ref[0:1] / c0
    mean1 = acc_ref[1:2] / c1
    ctr = yb - (m0 * mean0 + m1 * mean1) * ms_ref[...]

    @pl.when(ph == 1)
    def _():
        sq = ctr * ctr
        acc_ref[4:5] += jnp.sum(sq * m0, axis=0, keepdims=True)
        acc_ref[5:6] += jnp.sum(sq * m1, axis=0, keepdims=True)

    s0 = jnp.sqrt(acc_ref[4:5] / c0 + EPS)
    s1 = jnp.sqrt(acc_ref[5:6] / c1 + EPS)
    o = w_ref[...] * ctr / (m0 * s0 + m1 * s1) + b_ref[...]
    o_ref[0] = jnp.maximum(o, 0.0)


def _gnorm(y, bcol, w, b, ms):
    T_, NN_, H_ = y.shape
    BN = 2000
    return pl.pallas_call(
        _gnorm_body,
        grid=(T_, 3, NN_ // BN),
        in_specs=[
            pl.BlockSpec((1, BN, H_), lambda t, p, n: (t, n, 0)),
            pl.BlockSpec((BN, 1), lambda t, p, n: (n, 0)),
            pl.BlockSpec((1, H_), lambda t, p, n: (0, 0)),
            pl.BlockSpec((1, H_), lambda t, p, n: (0, 0)),
            pl.BlockSpec((1, H_), lambda t, p, n: (0, 0)),
        ],
        out_specs=pl.BlockSpec((1, BN, H_), lambda t, p, n: (t, n, 0)),
        out_shape=jax.ShapeDtypeStruct((T_, NN_, H_), jnp.float32),
        scratch_shapes=[pltpu.VMEM((8, H_), jnp.float32)],
    )(y, bcol, w, b, ms)


def _gat_sc_body(h_hbm, d_hbm, src_hbm, dst_hbm, zeros_hbm, bias_hbm,
                 out_hbm, d_v, dst_a, gidx_a, sc_a, dst_b, gidx_b, sc_b,
                 exbuf, obuf, b_v, num_sh, sem_a, sem_b):
    c = lax.axis_index("c")
    sid = lax.axis_index("s")
    pltpu.sync_copy(bias_hbm.at[pl.ds(c * HH, HH)], b_v)
    nzch = (NZCH - sid + 15) // 16
    nwch = (NWCH - sid + 15) // 16
    lanes = lax.iota(jnp.int32, 16)
    col64 = jnp.full((16,), HH, jnp.int32)
    nblk = NE // 16 // EK

    def t_body(t, carry):
        tc = t * 2 + c
        row_off = tc * NN

        def z_body(i, _):
            nb = (sid + i * 16) * ZCH
            pltpu.sync_copy(zeros_hbm, num_sh.at[pl.ds(nb, ZCH)])
            return 0
        lax.fori_loop(0, nzch, z_body, 0)
        pltpu.sync_copy(d_hbm.at[pl.ds(t * NN, NN)], d_v)
        plsc.subcore_barrier()

        def start(blk, dst_x, gidx_x, sc_x, sem_x):
            base = sid * (NE // 16) + (blk % nblk) * EK
            pltpu.sync_copy(src_hbm.at[pl.ds(base, EK)], gidx_x)
            pltpu.sync_copy(dst_hbm.at[pl.ds(base, EK)], dst_x)
            for q in range(EK // 16):
                sl = pl.ds(q * 16, 16)
                gidx_x[sl] = gidx_x[sl] + row_off
            pltpu.make_async_copy(h_hbm.at[gidx_x], sc_x, sem_x).start()

        def finish(dst_x, gidx_x, sc_x, sem_x):
            pltpu.make_async_copy(h_hbm.at[gidx_x], sc_x, sem_x).wait()
            for q in range(EK // 16):
                sl = pl.ds(q * 16, 16)
                rows = lanes + (q * 16)
                sv = plsc.load_gather(sc_x, [rows, col64])
                dv = plsc.load_gather(d_v, [dst_x[sl]])
                al = sv + dv
                al = jnp.where(al >= 0.0, al, al * NEG)
                exbuf[sl] = jnp.exp(al)
            for q in range(EK // 16):
                exq = exbuf[pl.ds(q * 16, 16)]
                for l in range(16):
                    e = q * 16 + l
                    exs = jnp.full((16,), exq[l], jnp.float32)
                    for j in range(4):
                        sj = pl.ds(j * 16, 16)
                        sc_x[e, sj] = sc_x[e, sj] * exs
                    v56 = sc_x[e, pl.ds(56, 16)]
                    sc_x[e, pl.ds(56, 16)] = jnp.where(
                        lanes < 8, v56,
                        jnp.where(lanes == 8, exs, jnp.zeros((16,), jnp.float32)))
            pltpu.sync_copy(sc_x, num_sh.at[dst_x], add=True)

        start(0, dst_a, gidx_a, sc_a, sem_a)

        def e_body(k, _):
            start(2 * k + 1, dst_b, gidx_b, sc_b, sem_b)
            finish(dst_a, gidx_a, sc_a, sem_a)
            start(2 * k + 2, dst_a, gidx_a, sc_a, sem_a)
            finish(dst_b, gidx_b, sc_b, sem_b)
            return 0
        lax.fori_loop(0, nblk // 2, e_body, 0)
        pltpu.make_async_copy(h_hbm.at[gidx_a], sc_a, sem_a).wait()
        plsc.subcore_barrier()

        def w_body(i, _):
            nb = (sid + i * 16) * WCH
            pltpu.sync_copy(num_sh.at[pl.ds(nb, WCH)], sc_a.at[pl.ds(0, WCH)])
            for e in range(WCH):
                denv = sc_a[e, pl.ds(56, 16)]
                dens = jnp.full((16,), denv[8] + 1e-16, jnp.float32)
                for j in range(4):
                    sj = pl.ds(j * 16, 16)
                    obuf[e, sj] = sc_a[e, sj] / dens + b_v[sj]
            pltpu.sync_copy(obuf, out_hbm.at[pl.ds(row_off + nb, WCH)])
            return 0
        lax.fori_loop(0, nwch, w_body, 0)
        plsc.subcore_barrier()
        return 0

    lax.fori_loop(0, NT, t_body, 0)


_gat_sc = functools.partial(
    pl.kernel,
    mesh=plsc.VectorSubcoreMesh(core_axis_name="c", subcore_axis_name="s"),
    out_type=jax.ShapeDtypeStruct((2 * NT * NN, HH), jnp.float32),
    compiler_params=pltpu.CompilerParams(
        needs_layout_passes=False, use_tc_tiling_on_sc=False),
    scratch_types=[
        pltpu.VMEM((NN,), jnp.float32),
        pltpu.VMEM((EK,), jnp.int32),
        pltpu.VMEM((EK,), jnp.int32),
        pltpu.VMEM((EK, WACC), jnp.float32),
        pltpu.VMEM((EK,), jnp.int32),
        pltpu.VMEM((EK,), jnp.int32),
        pltpu.VMEM((EK, WACC), jnp.float32),
        pltpu.VMEM((EK,), jnp.float32),
        pltpu.VMEM((WCH, HH), jnp.float32),
        pltpu.VMEM((HH,), jnp.float32),
        pltpu.VMEM_SHARED((NN, WACC), jnp.float32),
        pltpu.SemaphoreType.DMA,
        pltpu.SemaphoreType.DMA,
    ],
)(_gat_sc_body)


def kernel(x, batch, edge_index, Wp1, bp1, Wq1, bq1, Wr1, br1, gat_W, att_src,
           att_dst, gat_b, gn_w, gn_b, gn_ms, Wp2, bp2, Wq2, bq2, Wr2, br2):
    out0 = _tconv(x, Wp1, bp1, Wq1, bq1, Wr1, br1)
    b_, t_, n_, h_ = out0.shape

    xg = out0.transpose(1, 0, 2, 3).reshape(t_, b_ * n_, h_)
    asd = jnp.zeros((h_, 16), jnp.float32).at[:, 0].set(att_src).at[:, 1].set(att_dst)
    h_all, aux = _pre(xg, gat_W, asd)
    d_all = aux[:, :, 1].reshape(-1)
    h_flat = h_all.reshape(2 * t_ * b_ * n_, WACC)
    src = edge_index[0].astype(jnp.int32)
    dst = edge_index[1].astype(jnp.int32)
    zeros = jnp.zeros((ZCH, WACC), jnp.float32)

    y = _gat_sc(h_flat, d_all, src, dst, zeros, gat_b)
    y = y.reshape(t_, 2, b_ * n_, HH).transpose(0, 2, 1, 3).reshape(t_, b_ * n_, h_)

    bcol = batch.astype(jnp.float32).reshape(b_ * n_, 1)
    y = _gnorm(y, bcol, gn_w.reshape(1, h_), gn_b.reshape(1, h_), gn_ms.reshape(1, h_))
    y = y.reshape(t_, b_, n_, h_).transpose(1, 0, 2, 3)
    return _tconv(y, Wp2, bp2, Wq2, bq2, Wr2, br2)


# speedup vs baseline: 15.0472x; 1.1998x over previous
import functools

import jax
import jax.numpy as jnp
from jax import lax
from jax.experimental import pallas as pl
from jax.experimental.pallas import tpu as pltpu
from jax.experimental.pallas import tpu_sc as plsc

NEG = 0.2
EPS = 1e-5

NN = 20000
NE = 320000
NT = 6
HH = 64
WACC = 72
EK = 80
WCH = 40
NWCH = NN // WCH
ZCH = 160
NZCH = NN // ZCH


def _tconv_body(Tout, Kk, x_ref, wp_ref, bp_ref, wq_ref, bq_ref, wr_ref, br_ref, o_ref):
    xb = x_ref[0]
    for to in range(Tout):
        p = bp_ref[...].astype(jnp.float32)
        q = bq_ref[...].astype(jnp.float32)
        r = br_ref[...].astype(jnp.float32)
        for k in range(Kk):
            xt = xb[to + k]
            p = p + jnp.dot(xt, wp_ref[k], preferred_element_type=jnp.float32)
            q = q + jnp.dot(xt, wq_ref[k], preferred_element_type=jnp.float32)
            r = r + jnp.dot(xt, wr_ref[k], preferred_element_type=jnp.float32)
        o_ref[0, to] = jnp.maximum(p * jax.nn.sigmoid(q) + r, 0.0)


def _tconv(x, Wp, bp, Wq, bq, Wr, br):
    B_, Tin, N_, C = x.shape
    Kk, _, H_ = Wp.shape
    Tout = Tin - Kk + 1
    BN = 1000
    grid = (B_, N_ // BN)
    return pl.pallas_call(
        functools.partial(_tconv_body, Tout, Kk),
        grid=grid,
        in_specs=[
            pl.BlockSpec((1, Tin, BN, C), lambda b, n: (b, 0, n, 0)),
            pl.BlockSpec((Kk, C, H_), lambda b, n: (0, 0, 0)),
            pl.BlockSpec((H_,), lambda b, n: (0,)),
            pl.BlockSpec((Kk, C, H_), lambda b, n: (0, 0, 0)),
            pl.BlockSpec((H_,), lambda b, n: (0,)),
            pl.BlockSpec((Kk, C, H_), lambda b, n: (0, 0, 0)),
            pl.BlockSpec((H_,), lambda b, n: (0,)),
        ],
        out_specs=pl.BlockSpec((1, Tout, BN, H_), lambda b, n: (b, 0, n, 0)),
        out_shape=jax.ShapeDtypeStruct((B_, Tout, N_, H_), jnp.float32),
    )(x, Wp, bp, Wq, bq, Wr, br)


def _pre_body(x_ref, w_ref, asd_ref, h_ref, aux_ref):
    xb = x_ref[0]
    h = jnp.dot(xb, w_ref[...], preferred_element_type=jnp.float32)
    aux = jnp.dot(h, asd_ref[...], preferred_element_type=jnp.float32)
    s = aux[:, 0:1]
    z = jnp.zeros((xb.shape[0], WACC - HH - 1), jnp.float32)
    h_ref[0, 0] = jnp.concatenate([h[:, :HH], s, z], axis=1)
    h_ref[0, 1] = jnp.concatenate([h[:, HH:], s, z], axis=1)
    aux_ref[0] = aux


def _pre(xg, W, asd):
    T_, NN_, C = xg.shape
    BN = 2000
    grid = (T_, NN_ // BN)
    return pl.pallas_call(
        _pre_body,
        grid=grid,
        in_specs=[
            pl.BlockSpec((1, BN, C), lambda t, n: (t, n, 0)),
            pl.BlockSpec((C, C), lambda t, n: (0, 0)),
            pl.BlockSpec((C, 16), lambda t, n: (0, 0)),
        ],
        out_specs=[
            pl.BlockSpec((1, 2, BN, WACC), lambda t, n: (t, 0, n, 0)),
            pl.BlockSpec((1, BN, 16), lambda t, n: (t, n, 0)),
        ],
        out_shape=[
            jax.ShapeDtypeStruct((T_, 2, NN_, WACC), jnp.float32),
            jax.ShapeDtypeStruct((T_, NN_, 16), jnp.float32),
        ],
    )(xg, W, asd)


def _gnorm_body(y_ref, bc_ref, w_ref, b_ref, ms_ref, o_ref, acc_ref):
    ph = pl.program_id(1)
    nb = pl.program_id(2)
    yb = y_ref[0]
    m1 = bc_ref[...]
    m0 = 1.0 - m1

    @pl.when((ph == 0) & (nb == 0))
    def _():
        acc_ref[...] = jnp.zeros_like(acc_ref)

    @pl.when(ph == 0)
    def _():
        acc_ref[0:1] += jnp.sum(yb * m0, axis=0, keepdims=True)
        acc_ref[1:2] += jnp.sum(yb * m1, axis=0, keepdims=True)
        acc_ref[2:3] += jnp.full((1, yb.shape[1]), 1.0) * jnp.sum(m0)
        acc_ref[3:4] += jnp.full((1, yb.shape[1]), 1.0) * jnp.sum(m1)

    c0 = jnp.maximum(acc_ref[2:3], 1.0)
    c1 = jnp.maximum(acc_ref[3:4], 1.0)
    mean0 = acc_ref[0:1] / c0
    mean1 = acc_ref[1:2] / c1
    ctr = yb - (m0 * mean0 + m1 * mean1) * ms_ref[...]

    @pl.when(ph == 1)
    def _():
        sq = ctr * ctr
        acc_ref[4:5] += jnp.sum(sq * m0, axis=0, keepdims=True)
        acc_ref[5:6] += jnp.sum(sq * m1, axis=0, keepdims=True)

    s0 = jnp.sqrt(acc_ref[4:5] / c0 + EPS)
    s1 = jnp.sqrt(acc_ref[5:6] / c1 + EPS)
    o = w_ref[...] * ctr / (m0 * s0 + m1 * s1) + b_ref[...]
    o_ref[0] = jnp.maximum(o, 0.0)


def _gnorm(y, bcol, w, b, ms):
    T_, NN_, H_ = y.shape
    BN = 2000
    return pl.pallas_call(
        _gnorm_body,
        grid=(T_, 3, NN_ // BN),
        in_specs=[
            pl.BlockSpec((1, BN, H_), lambda t, p, n: (t, n, 0)),
            pl.BlockSpec((BN, 1), lambda t, p, n: (n, 0)),
            pl.BlockSpec((1, H_), lambda t, p, n: (0, 0)),
            pl.BlockSpec((1, H_), lambda t, p, n: (0, 0)),
            pl.BlockSpec((1, H_), lambda t, p, n: (0, 0)),
        ],
        out_specs=pl.BlockSpec((1, BN, H_), lambda t, p, n: (t, n, 0)),
        out_shape=jax.ShapeDtypeStruct((T_, NN_, H_), jnp.float32),
        scratch_shapes=[pltpu.VMEM((8, H_), jnp.float32)],
    )(y, bcol, w, b, ms)


def _gat_sc_body(h_hbm, d_hbm, pk_hbm, zeros_hbm, bias_hbm,
                 out_hbm, d_v, pk_a, gidx_a, sc_a, pk_b, gidx_b, sc_b,
                 exbuf, obuf, b_v, num_sh, sem_a, sem_b):
    c = lax.axis_index("c")
    sid = lax.axis_index("s")
    pltpu.sync_copy(bias_hbm.at[pl.ds(c * HH, HH)], b_v)
    nzch = (NZCH - sid + 15) // 16
    nwch = (NWCH - sid + 15) // 16
    lanes = lax.iota(jnp.int32, 16)
    col64 = jnp.full((16,), HH, jnp.int32)
    nblk = NE // 16 // EK

    def t_body(t, carry):
        tc = t * 2 + c
        row_off = tc * NN

        def z_body(i, _):
            nb = (sid + i * 16) * ZCH
            pltpu.sync_copy(zeros_hbm, num_sh.at[pl.ds(nb, ZCH)])
            return 0
        lax.fori_loop(0, nzch, z_body, 0)
        pltpu.sync_copy(d_hbm.at[pl.ds(t * NN, NN)], d_v)
        plsc.subcore_barrier()

        def start(blk, pk_x, gidx_x, sc_x, sem_x):
            bg = sid * nblk + (blk % nblk)
            pltpu.sync_copy(pk_hbm.at[bg], pk_x)
            for q in range(EK // 16):
                sl = pl.ds(q * 16, 16)
                gidx_x[sl] = pk_x[0, sl] + row_off
            pltpu.make_async_copy(h_hbm.at[gidx_x], sc_x, sem_x).start()

        def finish(pk_x, gidx_x, sc_x, sem_x):
            pltpu.make_async_copy(h_hbm.at[gidx_x], sc_x, sem_x).wait()
            for q in range(EK // 16):
                sl = pl.ds(q * 16, 16)
                rows = lanes + (q * 16)
                sv = plsc.load_gather(sc_x, [rows, col64])
                dv = plsc.load_gather(d_v, [pk_x[1, sl]])
                al = sv + dv
                al = jnp.where(al >= 0.0, al, al * NEG)
                exbuf[sl] = jnp.exp(al)
            for q in range(EK // 16):
                exq = exbuf[pl.ds(q * 16, 16)]
                for l in range(16):
                    e = q * 16 + l
                    exs = jnp.full((16,), exq[l], jnp.float32)
                    for j in range(4):
                        sj = pl.ds(j * 16, 16)
                        sc_x[e, sj] = sc_x[e, sj] * exs
                    v56 = sc_x[e, pl.ds(56, 16)]
                    sc_x[e, pl.ds(56, 16)] = jnp.where(
                        lanes < 8, v56,
                        jnp.where(lanes == 8, exs, jnp.zeros((16,), jnp.float32)))
            pltpu.sync_copy(sc_x, num_sh.at[pk_x.at[1]], add=True)

        start(0, pk_a, gidx_a, sc_a, sem_a)

        def e_body(k, _):
            start(2 * k + 1, pk_b, gidx_b, sc_b, sem_b)
            finish(pk_a, gidx_a, sc_a, sem_a)
            start(2 * k + 2, pk_a, gidx_a, sc_a, sem_a)
            finish(pk_b, gidx_b, sc_b, sem_b)
            return 0
        lax.fori_loop(0, nblk // 2, e_body, 0)
        pltpu.make_async_copy(h_hbm.at[gidx_a], sc_a, sem_a).wait()
        plsc.subcore_barrier()

        def w_body(i, _):
            nb = (sid + i * 16) * WCH
            pltpu.sync_copy(num_sh.at[pl.ds(nb, WCH)], sc_a.at[pl.ds(0, WCH)])
            for e in range(WCH):
                denv = sc_a[e, pl.ds(56, 16)]
                dens = jnp.full((16,), denv[8] + 1e-16, jnp.float32)
                for j in range(4):
                    sj = pl.ds(j * 16, 16)
                    obuf[e, sj] = sc_a[e, sj] / dens + b_v[sj]
            pltpu.sync_copy(obuf, out_hbm.at[pl.ds(row_off + nb, WCH)])
            return 0
        lax.fori_loop(0, nwch, w_body, 0)
        plsc.subcore_barrier()
        return 0

    lax.fori_loop(0, NT, t_body, 0)


_gat_sc = functools.partial(
    pl.kernel,
    mesh=plsc.VectorSubcoreMesh(core_axis_name="c", subcore_axis_name="s"),
    out_type=jax.ShapeDtypeStruct((2 * NT * NN, HH), jnp.float32),
    compiler_params=pltpu.CompilerParams(
        needs_layout_passes=False, use_tc_tiling_on_sc=False),
    scratch_types=[
        pltpu.VMEM((NN,), jnp.float32),
        pltpu.VMEM((2, EK), jnp.int32),
        pltpu.VMEM((EK,), jnp.int32),
        pltpu.VMEM((EK, WACC), jnp.float32),
        pltpu.VMEM((2, EK), jnp.int32),
        pltpu.VMEM((EK,), jnp.int32),
        pltpu.VMEM((EK, WACC), jnp.float32),
        pltpu.VMEM((EK,), jnp.float32),
        pltpu.VMEM((WCH, HH), jnp.float32),
        pltpu.VMEM((HH,), jnp.float32),
        pltpu.VMEM_SHARED((NN, WACC), jnp.float32),
        pltpu.SemaphoreType.DMA,
        pltpu.SemaphoreType.DMA,
    ],
)(_gat_sc_body)


def kernel(x, batch, edge_index, Wp1, bp1, Wq1, bq1, Wr1, br1, gat_W, att_src,
           att_dst, gat_b, gn_w, gn_b, gn_ms, Wp2, bp2, Wq2, bq2, Wr2, br2):
    out0 = _tconv(x, Wp1, bp1, Wq1, bq1, Wr1, br1)
    b_, t_, n_, h_ = out0.shape

    xg = out0.transpose(1, 0, 2, 3).reshape(t_, b_ * n_, h_)
    asd = jnp.zeros((h_, 16), jnp.float32).at[:, 0].set(att_src).at[:, 1].set(att_dst)
    h_all, aux = _pre(xg, gat_W, asd)
    d_all = aux[:, :, 1].reshape(-1)
    h_flat = h_all.reshape(2 * t_ * b_ * n_, WACC)
    src = edge_index[0].astype(jnp.int32)
    dst = edge_index[1].astype(jnp.int32)
    pk = jnp.stack([src.reshape(NE // EK, EK), dst.reshape(NE // EK, EK)], axis=1)
    zeros = jnp.zeros((ZCH, WACC), jnp.float32)

    y = _gat_sc(h_flat, d_all, pk, zeros, gat_b)
    y = y.reshape(t_, 2, b_ * n_, HH).transpose(0, 2, 1, 3).reshape(t_, b_ * n_, h_)

    bcol = batch.astype(jnp.float32).reshape(b_ * n_, 1)
    y = _gnorm(y, bcol, gn_w.reshape(1, h_), gn_b.reshape(1, h_), gn_ms.reshape(1, h_))
    y = y.reshape(t_, b_, n_, h_).transpose(1, 0, 2, 3)
    return _tconv(y, Wp2, bp2, Wq2, bq2, Wr2, br2)
